# Initial kernel scaffold; baseline (speedup 1.0000x reference)
import functools

import jax
import jax.numpy as jnp
from jax import lax
from jax.experimental import pallas as pl
from jax.experimental.pallas import tpu as pltpu
from jax.experimental.pallas import tpu_sc as plsc

VOCAB = 1000000
HIDDEN = 64
MAX_POS = 200
TYPE_VOCAB = 2
BATCH = 4096
SEQ = 200
EPS = 1e-12

L = 16
NC = 2
NS = 16
NW = NC * NS
TOK = BATCH * SEQ
TPW = TOK // NW
C = 512
NCH = TPW // C
JROWS = C // 128


def _rsqrt_f32(x):
    i = lax.bitcast_convert_type(x, jnp.int32)
    i = jnp.int32(0x5F3759DF) - lax.shift_right_logical(i, 1)
    y = lax.bitcast_convert_type(i, jnp.float32)
    for _ in range(4):
        y = y * (1.5 - 0.5 * x * y * y)
    return y


def _body(ids_hbm, pos_hbm, typ_hbm, word_hbm, postab_hbm, typtab_hbm,
          gb_hbm, out_hbm, idx_v, pids_v, tids_v, rows_v, postab_v,
          typtab_v, gb_v, sem):
    cid = lax.axis_index("c")
    sid = lax.axis_index("s")
    wid = sid * NC + cid
    base = wid * TPW
    row_base = wid * (TPW // 128)

    pltpu.sync_copy(postab_hbm, postab_v)
    pltpu.sync_copy(typtab_hbm, typtab_v)
    pltpu.sync_copy(gb_hbm, gb_v)
    g = [gb_v[0, pl.ds(k * L, L)] for k in range(4)]
    b = [gb_v[1, pl.ds(k * L, L)] for k in range(4)]

    def chunk(ci, carry):
        tb = base + ci * C
        pltpu.sync_copy(ids_hbm.at[pl.ds(row_base + ci * JROWS, JROWS)],
                        idx_v)
        pltpu.sync_copy(pos_hbm.at[pl.ds(tb, C)], pids_v)
        pltpu.sync_copy(typ_hbm.at[pl.ds(tb, C)], tids_v)
        cps = [pltpu.async_copy(word_hbm.at[idx_v.at[j]],
                                rows_v.at[pl.ds(j * 128, 128)], sem)
               for j in range(JROWS)]
        for cp in cps:
            cp.wait()

        def tok(t, acc):
            pid = pids_v[t]
            tid = tids_v[t]
            e = [rows_v[t, pl.ds(k * L, L)]
                 + postab_v[pid, pl.ds(k * L, L)]
                 + typtab_v[tid, pl.ds(k * L, L)]
                 for k in range(4)]
            s = jnp.sum(e[0] + e[1] + e[2] + e[3])
            q = jnp.sum(e[0] * e[0] + e[1] * e[1] + e[2] * e[2] + e[3] * e[3])
            mean = s * (1.0 / HIDDEN)
            var = q * (1.0 / HIDDEN) - mean * mean
            rs = _rsqrt_f32(var + EPS)
            c0 = -mean * rs
            for k in range(4):
                rows_v[t, pl.ds(k * L, L)] = (e[k] * rs + c0) * g[k] + b[k]
            return acc

        lax.fori_loop(0, C, tok, 0)
        pltpu.sync_copy(rows_v, out_hbm.at[pl.ds(tb, C)])
        return carry

    lax.fori_loop(0, NCH, chunk, 0)


@jax.jit
def _run(ids2d, pos_flat, typ_flat, word_table, pos_table, type_table, gb):
    mesh = plsc.VectorSubcoreMesh(core_axis_name="c", subcore_axis_name="s")
    f = pl.kernel(
        _body,
        out_type=jax.ShapeDtypeStruct((TOK, HIDDEN), jnp.float32),
        mesh=mesh,
        scratch_types=[
            pltpu.VMEM((JROWS, 128), jnp.int32),
            pltpu.VMEM((C,), jnp.int32),
            pltpu.VMEM((C,), jnp.int32),
            pltpu.VMEM((C, HIDDEN), jnp.float32),
            pltpu.VMEM((MAX_POS, HIDDEN), jnp.float32),
            pltpu.VMEM((TYPE_VOCAB, HIDDEN), jnp.float32),
            pltpu.VMEM((2, HIDDEN), jnp.float32),
            pltpu.SemaphoreType.DMA,
        ],
    )
    return f(ids2d, pos_flat, typ_flat, word_table, pos_table, type_table, gb)


def kernel(input_ids, position_ids, token_type_ids, word_table, pos_table,
           type_table, ln_gamma, ln_beta):
    ids2d = input_ids.reshape(TOK // 128, 128).astype(jnp.int32)
    pos_flat = position_ids.reshape(TOK).astype(jnp.int32)
    typ_flat = token_type_ids.reshape(TOK).astype(jnp.int32)
    gb = jnp.stack([ln_gamma, ln_beta]).astype(jnp.float32)
    out = _run(ids2d, pos_flat, typ_flat, word_table, pos_table, type_table,
               gb)
    return out.reshape(BATCH, SEQ, HIDDEN)


# speedup vs baseline: 2.4626x; 2.4626x over previous
import functools

import jax
import jax.numpy as jnp
from jax import lax
from jax.experimental import pallas as pl
from jax.experimental.pallas import tpu as pltpu
from jax.experimental.pallas import tpu_sc as plsc

VOCAB = 1000000
HIDDEN = 64
MAX_POS = 200
TYPE_VOCAB = 2
BATCH = 4096
SEQ = 200
EPS = 1e-12

L = 16
NC = 2
NS = 16
NW = NC * NS
TOK = BATCH * SEQ
TPW = TOK // NW
C = 512
NCH = TPW // C
JROWS = C // 128


def _rsqrt_f32(x):
    i = lax.bitcast_convert_type(x, jnp.int32)
    i = jnp.int32(0x5F3759DF) - lax.shift_right_logical(i, 1)
    y = lax.bitcast_convert_type(i, jnp.float32)
    for _ in range(4):
        y = y * (1.5 - 0.5 * x * y * y)
    return y


def _lane_perms():
    iota = lax.iota(jnp.int32, L)
    return [jnp.bitwise_xor(iota, jnp.int32(d)) for d in (1, 2, 4, 8)]


def _lanesum(v, perms):
    dnums = lax.GatherDimensionNumbers(
        offset_dims=(), collapsed_slice_dims=(0,), start_index_map=(0,))
    for p in perms:
        v = v + lax.gather(v, p[:, None], dnums, (1,),
                           unique_indices=True, indices_are_sorted=False,
                           mode=lax.GatherScatterMode.PROMISE_IN_BOUNDS)
    return v


def _body(ids_hbm, pos_hbm, typ_hbm, word_hbm, postab_hbm, typtab_hbm,
          gb_hbm, out_hbm, idx_v, pids_v, tids_v, rows_v, postab_v,
          typtab_v, gb_v, sem):
    cid = lax.axis_index("c")
    sid = lax.axis_index("s")
    wid = sid * NC + cid
    base = wid * TPW
    row_base = wid * (TPW // 128)

    pltpu.sync_copy(postab_hbm, postab_v)
    pltpu.sync_copy(typtab_hbm, typtab_v)
    pltpu.sync_copy(gb_hbm, gb_v)
    g = [gb_v[0, pl.ds(k * L, L)] for k in range(4)]
    b = [gb_v[1, pl.ds(k * L, L)] for k in range(4)]
    perms = _lane_perms()

    def chunk(ci, carry):
        tb = base + ci * C
        pltpu.sync_copy(ids_hbm.at[pl.ds(row_base + ci * JROWS, JROWS)],
                        idx_v)
        pltpu.sync_copy(pos_hbm.at[pl.ds(tb, C)], pids_v)
        pltpu.sync_copy(typ_hbm.at[pl.ds(tb, C)], tids_v)
        cps = [pltpu.async_copy(word_hbm.at[idx_v.at[j]],
                                rows_v.at[pl.ds(j * 128, 128)], sem)
               for j in range(JROWS)]
        for cp in cps:
            cp.wait()

        def group(gi, acc):
            pvec = pids_v[pl.ds(gi * L, L)]
            tvec = tids_v[pl.ds(gi * L, L)]
            for u in range(L):
                t = gi * L + u
                pid = pvec[u]
                tid = tvec[u]
                e = [rows_v[t, pl.ds(k * L, L)]
                     + postab_v[pid, pl.ds(k * L, L)]
                     + typtab_v[tid, pl.ds(k * L, L)]
                     for k in range(4)]
                s = _lanesum(e[0] + e[1] + e[2] + e[3], perms)
                q = _lanesum(e[0] * e[0] + e[1] * e[1]
                             + e[2] * e[2] + e[3] * e[3], perms)
                mean = s * (1.0 / HIDDEN)
                var = q * (1.0 / HIDDEN) - mean * mean
                rs = _rsqrt_f32(var + EPS)
                c0 = -mean * rs
                for k in range(4):
                    rows_v[t, pl.ds(k * L, L)] = (e[k] * rs + c0) * g[k] + b[k]
            return acc

        lax.fori_loop(0, C // L, group, 0)
        pltpu.sync_copy(rows_v, out_hbm.at[pl.ds(tb, C)])
        return carry

    lax.fori_loop(0, NCH, chunk, 0)


@jax.jit
def _run(ids2d, pos_flat, typ_flat, word_table, pos_table, type_table, gb):
    mesh = plsc.VectorSubcoreMesh(core_axis_name="c", subcore_axis_name="s")
    f = pl.kernel(
        _body,
        out_type=jax.ShapeDtypeStruct((TOK, HIDDEN), jnp.float32),
        mesh=mesh,
        scratch_types=[
            pltpu.VMEM((JROWS, 128), jnp.int32),
            pltpu.VMEM((C,), jnp.int32),
            pltpu.VMEM((C,), jnp.int32),
            pltpu.VMEM((C, HIDDEN), jnp.float32),
            pltpu.VMEM((MAX_POS, HIDDEN), jnp.float32),
            pltpu.VMEM((TYPE_VOCAB, HIDDEN), jnp.float32),
            pltpu.VMEM((2, HIDDEN), jnp.float32),
            pltpu.SemaphoreType.DMA,
        ],
        compiler_params=pltpu.CompilerParams(use_tc_tiling_on_sc=False),
    )
    return f(ids2d, pos_flat, typ_flat, word_table, pos_table, type_table, gb)


def kernel(input_ids, position_ids, token_type_ids, word_table, pos_table,
           type_table, ln_gamma, ln_beta):
    ids2d = input_ids.reshape(TOK // 128, 128).astype(jnp.int32)
    pos_flat = position_ids.reshape(TOK).astype(jnp.int32)
    typ_flat = token_type_ids.reshape(TOK).astype(jnp.int32)
    gb = jnp.stack([ln_gamma, ln_beta]).astype(jnp.float32)
    out = _run(ids2d, pos_flat, typ_flat, word_table, pos_table, type_table,
               gb)
    return out.reshape(BATCH, SEQ, HIDDEN)
